# baseline (speedup 1.0000x reference)
import functools

import jax
import jax.numpy as jnp
from jax import lax
from jax.experimental import pallas as pl
from jax.experimental.pallas import tpu as pltpu
from jax.experimental.pallas import tpu_sc as plsc

N = 10000
NPAD = 10240
E = 640000
NUM_GRAPHS = 64
D_IN = 128
D_EDGE = 16
H = 64
EMB = 128
L = 3

CHUNK = 128
NTILE = 16
NW = 2 * NTILE
CPW = 160
PADE = NW * CPW * CHUNK
ROWS_PER_TILE = NPAD // NTILE

def _sc_params():
    return pltpu.CompilerParams(use_tc_tiling_on_sc=False)


def _sc_agg_pre(hn, src3, dst3, edge_attr, z16, z1, ones_c, zb):
    mesh = plsc.VectorSubcoreMesh(core_axis_name="c", subcore_axis_name="s")

    NB = 4

    @functools.partial(
        pl.kernel,
        mesh=mesh,
        out_type=[
            jax.ShapeDtypeStruct((2, NPAD, H), jnp.bfloat16),
            jax.ShapeDtypeStruct((2, NPAD, D_EDGE), jnp.float32),
            jax.ShapeDtypeStruct((2, NPAD), jnp.float32),
        ],
        scratch_types=[
            pltpu.VMEM((CPW, CHUNK), jnp.int32),
            pltpu.VMEM((CPW, CHUNK), jnp.int32),
            [pltpu.VMEM((CHUNK, H), jnp.bfloat16)] * NB,
            [pltpu.VMEM((CHUNK, D_EDGE), jnp.float32)] * NB,
            pltpu.VMEM((CHUNK,), jnp.float32),
            pltpu.VMEM_SHARED((NPAD, H), jnp.bfloat16),
            pltpu.VMEM_SHARED((N, H), jnp.bfloat16),
            pltpu.VMEM_SHARED((NPAD, D_EDGE), jnp.float32),
            pltpu.VMEM_SHARED((NPAD,), jnp.float32),
            pltpu.SemaphoreType.DMA,
            pltpu.SemaphoreType.DMA,
            pltpu.SemaphoreType.DMA,
            pltpu.SemaphoreType.DMA,
            pltpu.SemaphoreType.DMA,
            pltpu.SemaphoreType.DMA,
            pltpu.SemaphoreType.DMA,
            [pltpu.SemaphoreType.DMA] * NB,
            [pltpu.SemaphoreType.DMA] * NB,
            [pltpu.SemaphoreType.DMA] * NB,
            [pltpu.SemaphoreType.DMA] * NB,
            [pltpu.SemaphoreType.DMA] * NB,
        ],
        compiler_params=_sc_params(),
    )
    def k(hn_hbm, src_hbm, dst_hbm, ea_hbm, z16_hbm, z1_hbm, ones_hbm, zb_hbm,
          gp_out, ea_out, deg_out,
          idxs, idxd, rows, attr, ones_v, acc, hn_sh, accE, accD,
          semz, semzE, semzD, semh, semi0, semi1, semo, sg, sa, ss, se, sd):
        c = lax.axis_index("c")
        s = lax.axis_index("s")
        w = c * NTILE + s
        r0 = s * ROWS_PER_TILE
        hrows = N // NTILE
        zc = pltpu.async_copy(zb_hbm.at[pl.ds(r0, ROWS_PER_TILE)],
                              acc.at[pl.ds(r0, ROWS_PER_TILE)], semz)
        zcE = pltpu.async_copy(z16_hbm.at[pl.ds(r0, ROWS_PER_TILE)],
                               accE.at[pl.ds(r0, ROWS_PER_TILE)], semzE)
        zcD = pltpu.async_copy(z1_hbm.at[pl.ds(r0, ROWS_PER_TILE)],
                               accD.at[pl.ds(r0, ROWS_PER_TILE)], semzD)
        hc = pltpu.async_copy(hn_hbm.at[pl.ds(s * hrows, hrows)],
                              hn_sh.at[pl.ds(s * hrows, hrows)], semh)
        ic0 = pltpu.async_copy(src_hbm.at[w], idxs, semi0)
        ic1 = pltpu.async_copy(dst_hbm.at[w], idxd, semi1)
        oc = pltpu.async_copy(ones_hbm, ones_v, semo)
        zc.wait()
        zcE.wait()
        zcD.wait()
        hc.wait()
        plsc.subcore_barrier()
        ic0.wait()
        ic1.wait()
        oc.wait()

        def lbase(kk):
            b = (w * CPW + kk) * CHUNK
            return jnp.where(b < E, b, 0)

        def g(kk, b):
            pltpu.async_copy(hn_sh.at[idxs.at[kk]], rows[b], sg[b])
            pltpu.async_copy(ea_hbm.at[pl.ds(lbase(kk), CHUNK)], attr[b],
                             sa[b])

        def gwait(kk, b):
            pltpu.make_async_copy(hn_sh.at[idxs.at[kk]], rows[b], sg[b]).wait()
            pltpu.make_async_copy(ea_hbm.at[pl.ds(lbase(kk), CHUNK)], attr[b],
                                  sa[b]).wait()

        def sca(kk, b):
            pltpu.async_copy(rows[b], acc.at[idxd.at[kk]], ss[b], add=True)
            pltpu.async_copy(attr[b], accE.at[idxd.at[kk]], se[b], add=True)
            pltpu.async_copy(ones_v, accD.at[idxd.at[kk]], sd[b], add=True)

        def swait(kk, b):
            pltpu.make_async_copy(rows[b], acc.at[idxd.at[kk]], ss[b]).wait()
            pltpu.make_async_copy(attr[b], accE.at[idxd.at[kk]], se[b]).wait()
            pltpu.make_async_copy(ones_v, accD.at[idxd.at[kk]], sd[b]).wait()

        for b in range(NB):
            g(b, b)

        def step(j, carry):
            k0 = NB * j
            for b in range(NB):
                gwait(k0 + b, b)
                sca(k0 + b, b)
            for b in range(NB):
                swait(k0 + b, b)
                g(k0 + NB + b, b)
            return carry

        lax.fori_loop(0, CPW // NB - 1, step, 0)
        k0 = CPW - NB
        for b in range(NB):
            gwait(k0 + b, b)
            sca(k0 + b, b)
        for b in range(NB):
            swait(k0 + b, b)

        plsc.subcore_barrier()
        pltpu.sync_copy(acc.at[pl.ds(r0, ROWS_PER_TILE)],
                        gp_out.at[c, pl.ds(r0, ROWS_PER_TILE)])
        pltpu.sync_copy(accE.at[pl.ds(r0, ROWS_PER_TILE)],
                        ea_out.at[c, pl.ds(r0, ROWS_PER_TILE)])
        pltpu.sync_copy(accD.at[pl.ds(r0, ROWS_PER_TILE)],
                        deg_out.at[c, pl.ds(r0, ROWS_PER_TILE)])

    return k(hn, src3, dst3, edge_attr, z16, z1, ones_c, zb)


def _sc_agg(hn, src3, dst3, zb):
    mesh = plsc.VectorSubcoreMesh(core_axis_name="c", subcore_axis_name="s")

    NB = 4

    @functools.partial(
        pl.kernel,
        mesh=mesh,
        out_type=jax.ShapeDtypeStruct((2, NPAD, H), jnp.bfloat16),
        scratch_types=[
            pltpu.VMEM((CPW, CHUNK), jnp.int32),
            pltpu.VMEM((CPW, CHUNK), jnp.int32),
            [pltpu.VMEM((CHUNK, H), jnp.bfloat16)] * NB,
            pltpu.VMEM_SHARED((NPAD, H), jnp.bfloat16),
            pltpu.VMEM_SHARED((N, H), jnp.bfloat16),
            pltpu.SemaphoreType.DMA,
            pltpu.SemaphoreType.DMA,
            pltpu.SemaphoreType.DMA,
            pltpu.SemaphoreType.DMA,
            [pltpu.SemaphoreType.DMA] * NB,
            [pltpu.SemaphoreType.DMA] * NB,
        ],
        compiler_params=_sc_params(),
    )
    def k(hn_hbm, src_hbm, dst_hbm, z_hbm, out_hbm,
          idxs, idxd, rows, acc, hn_sh,
          semz, semh, semi0, semi1, sg, ss):
        c = lax.axis_index("c")
        s = lax.axis_index("s")
        w = c * NTILE + s
        r0 = s * ROWS_PER_TILE
        hrows = N // NTILE
        zc = pltpu.async_copy(z_hbm.at[pl.ds(r0, ROWS_PER_TILE)],
                              acc.at[pl.ds(r0, ROWS_PER_TILE)], semz)
        hc = pltpu.async_copy(hn_hbm.at[pl.ds(s * hrows, hrows)],
                              hn_sh.at[pl.ds(s * hrows, hrows)], semh)
        ic0 = pltpu.async_copy(src_hbm.at[w], idxs, semi0)
        ic1 = pltpu.async_copy(dst_hbm.at[w], idxd, semi1)
        zc.wait()
        hc.wait()
        plsc.subcore_barrier()
        ic0.wait()
        ic1.wait()

        def g(kk, b):
            pltpu.async_copy(hn_sh.at[idxs.at[kk]], rows[b], sg[b])

        def gwait(kk, b):
            pltpu.make_async_copy(hn_sh.at[idxs.at[kk]], rows[b], sg[b]).wait()

        def sca(kk, b):
            pltpu.async_copy(rows[b], acc.at[idxd.at[kk]], ss[b], add=True)

        def swait(kk, b):
            pltpu.make_async_copy(rows[b], acc.at[idxd.at[kk]], ss[b]).wait()

        for b in range(NB):
            g(b, b)

        def step(j, carry):
            k0 = NB * j
            for b in range(NB):
                gwait(k0 + b, b)
                sca(k0 + b, b)
            for b in range(NB):
                swait(k0 + b, b)
                g(k0 + NB + b, b)
            return carry

        lax.fori_loop(0, CPW // NB - 1, step, 0)
        k0 = CPW - NB
        for b in range(NB):
            gwait(k0 + b, b)
            sca(k0 + b, b)
        for b in range(NB):
            swait(k0 + b, b)

        plsc.subcore_barrier()
        pltpu.sync_copy(acc.at[pl.ds(r0, ROWS_PER_TILE)],
                        out_hbm.at[c, pl.ds(r0, ROWS_PER_TILE)])

    return k(hn, src3, dst3, zb)


NROWS = E // CHUNK
NROWS_PAD = NW * CPW - NROWS


def _ln(h, g, b):
    mu = jnp.mean(h, axis=-1, keepdims=True)
    var = jnp.mean((h - mu) ** 2, axis=-1, keepdims=True)
    return (h - mu) / jnp.sqrt(var + 1e-5) * g + b


def _pad_idx_body(ei_ref, so_ref, do_ref):
    pad_s = jnp.zeros((NROWS_PAD, CHUNK), jnp.int32)
    pad_d = jnp.full((NROWS_PAD, CHUNK), N, jnp.int32)
    so_ref[...] = jnp.concatenate([ei_ref[0], pad_s], axis=0)
    do_ref[...] = jnp.concatenate([ei_ref[1], pad_d], axis=0)


def _bea_body(eap_ref, degp_ref, wme_ref, bm_ref, o0_ref, o1_ref, o2_ref):
    ea = eap_ref[0, :N] + eap_ref[1, :N]
    deg = degp_ref[...]
    for i, o_ref in enumerate((o0_ref, o1_ref, o2_ref)):
        o_ref[...] = (jnp.dot(ea, wme_ref[i],
                              preferred_element_type=jnp.float32)
                      + deg * bm_ref[i])


def _q_body(hnb_ref, ws_ref, bs_ref, o_ref):
    o_ref[...] = (jnp.dot(hnb_ref[...], ws_ref[...],
                          preferred_element_type=jnp.float32)
                  + bs_ref[...])


def _layer_update(h_ref, t_ref, gp_ref, wmh_ref, bea_ref):
    g = (gp_ref[0, :N].astype(jnp.float32)
         + gp_ref[1, :N].astype(jnp.float32))
    up = (t_ref[...] + bea_ref[...]
          + jnp.dot(g, wmh_ref[...], preferred_element_type=jnp.float32))
    return h_ref[...] + jnp.maximum(up, 0.0)


def _first_body(x_ref, w_ref, b_ref, g_ref, be_ref, h_ref, hnb_ref):
    h = jnp.dot(x_ref[...], w_ref[...],
                preferred_element_type=jnp.float32) + b_ref[...]
    h_ref[...] = h
    hnb_ref[...] = _ln(h, g_ref[...], be_ref[...]).astype(jnp.bfloat16)


def _upd_body(h_ref, t_ref, gp_ref, wmh_ref, bea_ref, g2_ref, be2_ref,
              ws2_ref, bs2_ref,
              ho_ref, hnbo_ref, t2_ref):
    h = _layer_update(h_ref, t_ref, gp_ref, wmh_ref, bea_ref)
    ho_ref[...] = h
    hn = _ln(h, g2_ref[...], be2_ref[...])
    hnbo_ref[...] = hn.astype(jnp.bfloat16)
    t2_ref[...] = (jnp.dot(hn, ws2_ref[...],
                           preferred_element_type=jnp.float32)
                   + bs2_ref[...])


def _final_body(h_ref, t_ref, gp_ref, wmh_ref, bea_ref, batch_ref,
                w1_ref, b1_ref, w2_ref, b2_ref, o_ref):
    h = _layer_update(h_ref, t_ref, gp_ref, wmh_ref, bea_ref)
    gids = lax.broadcasted_iota(jnp.int32, (1, NUM_GRAPHS), 1)
    onehot = (batch_ref[...] == gids).astype(jnp.float32)
    pooled = lax.dot_general(onehot, h, (((0,), (0,)), ((), ())),
                             preferred_element_type=jnp.float32)
    counts = lax.dot_general(onehot, jnp.ones((N, 1), jnp.float32),
                             (((0,), (0,)), ((), ())),
                             preferred_element_type=jnp.float32)
    pooled = pooled / jnp.maximum(counts, 1.0)
    e = jnp.maximum(
        jnp.dot(pooled, w1_ref[...], preferred_element_type=jnp.float32)
        + b1_ref[...], 0.0)
    e = jnp.dot(e, w2_ref[...], preferred_element_type=jnp.float32) + b2_ref[...]
    norm = jnp.sqrt(jnp.sum(e * e, axis=-1, keepdims=True))
    o_ref[...] = e / jnp.maximum(norm, 1e-12)


def _tc(body, out_shape, *args):
    return pl.pallas_call(body, out_shape=out_shape)(*args)


def kernel(x, edge_index, edge_attr, batch, Win, bin_, gamma, beta,
           Wm, bm, Ws, bs, W1, b1, W2, b2):
    f32 = jnp.float32
    ei3 = edge_index.reshape(2, NROWS, CHUNK)
    src2, dst2 = _tc(
        _pad_idx_body,
        [jax.ShapeDtypeStruct((NW * CPW, CHUNK), jnp.int32),
         jax.ShapeDtypeStruct((NW * CPW, CHUNK), jnp.int32)],
        ei3)
    src3 = src2.reshape(NW, CPW, CHUNK)
    dst3 = dst2.reshape(NW, CPW, CHUNK)
    zb = jnp.zeros((NPAD, H), jnp.bfloat16)
    z16 = jnp.zeros((NPAD, D_EDGE), f32)
    z1 = jnp.zeros((NPAD,), f32)
    ones_c = jnp.ones((CHUNK,), f32)

    h, hnb = _tc(
        _first_body,
        [jax.ShapeDtypeStruct((N, H), f32),
         jax.ShapeDtypeStruct((N, H), jnp.bfloat16)],
        x, Win, bin_.reshape(1, H),
        gamma[0].reshape(1, H), beta[0].reshape(1, H))

    nh = jax.ShapeDtypeStruct((N, H), f32)
    t = _tc(_q_body, nh, hnb, Ws[0], bs[0].reshape(1, H))

    for i in range(L):
        wm_h = Wm[i][:H]
        if i == 0:
            gp, eap_pad, degp_pad = _sc_agg_pre(
                hnb, src3, dst3, edge_attr, z16, z1, ones_c, zb)
            degp = (degp_pad[0, :N] + degp_pad[1, :N]).reshape(N, 1)
            bea = _tc(_bea_body, [nh, nh, nh],
                      eap_pad, degp, Wm[:, H:, :], bm.reshape(L, 1, H))
        else:
            gp = _sc_agg(hnb, src3, dst3, zb)
        if i < L - 1:
            h, hnb, t = _tc(
                _upd_body,
                [nh, jax.ShapeDtypeStruct((N, H), jnp.bfloat16), nh],
                h, t, gp, wm_h, bea[i],
                gamma[i + 1].reshape(1, H), beta[i + 1].reshape(1, H),
                Ws[i + 1], bs[i + 1].reshape(1, H))
        else:
            out = _tc(
                _final_body, jax.ShapeDtypeStruct((NUM_GRAPHS, EMB), f32),
                h, t, gp, wm_h, bea[i],
                batch.reshape(N, 1), W1, b1.reshape(1, EMB),
                W2, b2.reshape(1, EMB))
    return out


# speedup vs baseline: 1.1485x; 1.1485x over previous
import functools

import jax
import jax.numpy as jnp
from jax import lax
from jax.experimental import pallas as pl
from jax.experimental.pallas import tpu as pltpu
from jax.experimental.pallas import tpu_sc as plsc

N = 10000
NPAD = 10240
E = 640000
NUM_GRAPHS = 64
D_IN = 128
D_EDGE = 16
H = 64
EMB = 128
L = 3

CHUNK = 128
NTILE = 16
NW = 2 * NTILE
CPW = 160
PADE = NW * CPW * CHUNK
ROWS_PER_TILE = NPAD // NTILE

def _sc_params():
    return pltpu.CompilerParams(use_tc_tiling_on_sc=False)


def _sc_edge_pre(edge_attr, dst3, z16, z1, ones_c):
    mesh = plsc.VectorSubcoreMesh(core_axis_name="c", subcore_axis_name="s")

    @functools.partial(
        pl.kernel,
        mesh=mesh,
        out_type=[
            jax.ShapeDtypeStruct((2, NPAD, D_EDGE), jnp.float32),
            jax.ShapeDtypeStruct((2, NPAD), jnp.float32),
        ],
        scratch_types=[
            pltpu.VMEM((CPW, CHUNK), jnp.int32),
            pltpu.VMEM((CHUNK, D_EDGE), jnp.float32),
            pltpu.VMEM((CHUNK, D_EDGE), jnp.float32),
            pltpu.VMEM((CHUNK,), jnp.float32),
            pltpu.VMEM_SHARED((NPAD, D_EDGE), jnp.float32),
            pltpu.VMEM_SHARED((NPAD,), jnp.float32),
        ] + [pltpu.SemaphoreType.DMA] * 10,
        compiler_params=_sc_params(),
    )
    def k(ea_hbm, dst_hbm, z16_hbm, z1_hbm, ones_hbm, ea_out, deg_out,
          idxd, attr0, attr1, ones_v, accE, accD,
          semz0, semz1, semi, semo, sa0, sa1, se0, se1, sd0, sd1):
        c = lax.axis_index("c")
        s = lax.axis_index("s")
        w = c * NTILE + s
        r0 = s * ROWS_PER_TILE
        zc0 = pltpu.async_copy(z16_hbm.at[pl.ds(r0, ROWS_PER_TILE)],
                               accE.at[pl.ds(r0, ROWS_PER_TILE)], semz0)
        zc1 = pltpu.async_copy(z1_hbm.at[pl.ds(r0, ROWS_PER_TILE)],
                               accD.at[pl.ds(r0, ROWS_PER_TILE)], semz1)
        ic = pltpu.async_copy(dst_hbm.at[w], idxd, semi)
        oc = pltpu.async_copy(ones_hbm, ones_v, semo)
        zc0.wait()
        zc1.wait()
        plsc.subcore_barrier()
        ic.wait()
        oc.wait()

        def lbase(kk):
            b = (w * CPW + kk) * CHUNK
            return jnp.where(b < E, b, 0)

        def al(kk, buf, sem):
            pltpu.async_copy(ea_hbm.at[pl.ds(lbase(kk), CHUNK)], buf, sem)

        def alwait(kk, buf, sem):
            pltpu.make_async_copy(ea_hbm.at[pl.ds(lbase(kk), CHUNK)], buf,
                                  sem).wait()

        def fire(kk, buf, sem_e, sem_d):
            pltpu.async_copy(buf, accE.at[idxd.at[kk]], sem_e, add=True)
            pltpu.async_copy(ones_v, accD.at[idxd.at[kk]], sem_d, add=True)

        def ewait(kk, buf, sem):
            pltpu.make_async_copy(buf, accE.at[idxd.at[kk]], sem).wait()

        def dwait(kk, sem):
            pltpu.make_async_copy(ones_v, accD.at[idxd.at[kk]], sem).wait()

        al(0, attr0, sa0)
        al(1, attr1, sa1)

        def step(j, carry):
            k0 = 2 * j
            k1 = k0 + 1
            alwait(k0, attr0, sa0)
            fire(k0, attr0, se0, sd0)
            alwait(k1, attr1, sa1)
            fire(k1, attr1, se1, sd1)
            ewait(k0, attr0, se0)
            al(k0 + 2, attr0, sa0)
            ewait(k1, attr1, se1)
            al(k1 + 2, attr1, sa1)
            dwait(k0, sd0)
            dwait(k1, sd1)
            return carry

        lax.fori_loop(0, CPW // 2 - 1, step, 0)
        k0 = CPW - 2
        k1 = CPW - 1
        alwait(k0, attr0, sa0)
        fire(k0, attr0, se0, sd0)
        alwait(k1, attr1, sa1)
        fire(k1, attr1, se1, sd1)
        ewait(k0, attr0, se0)
        ewait(k1, attr1, se1)
        dwait(k0, sd0)
        dwait(k1, sd1)

        plsc.subcore_barrier()
        pltpu.sync_copy(accE.at[pl.ds(r0, ROWS_PER_TILE)],
                        ea_out.at[c, pl.ds(r0, ROWS_PER_TILE)])
        pltpu.sync_copy(accD.at[pl.ds(r0, ROWS_PER_TILE)],
                        deg_out.at[c, pl.ds(r0, ROWS_PER_TILE)])

    return k(edge_attr, dst3, z16, z1, ones_c)


def _sc_agg(hn, src3, dst3, zb):
    mesh = plsc.VectorSubcoreMesh(core_axis_name="c", subcore_axis_name="s")

    NB = 8

    @functools.partial(
        pl.kernel,
        mesh=mesh,
        out_type=jax.ShapeDtypeStruct((2, NPAD, H), jnp.bfloat16),
        scratch_types=[
            pltpu.VMEM((CPW, CHUNK), jnp.int32),
            pltpu.VMEM((CPW, CHUNK), jnp.int32),
            [pltpu.VMEM((CHUNK, H), jnp.bfloat16)] * NB,
            pltpu.VMEM_SHARED((NPAD, H), jnp.bfloat16),
            pltpu.VMEM_SHARED((N, H), jnp.bfloat16),
            pltpu.SemaphoreType.DMA,
            pltpu.SemaphoreType.DMA,
            pltpu.SemaphoreType.DMA,
            pltpu.SemaphoreType.DMA,
            [pltpu.SemaphoreType.DMA] * NB,
            [pltpu.SemaphoreType.DMA] * NB,
        ],
        compiler_params=_sc_params(),
    )
    def k(hn_hbm, src_hbm, dst_hbm, z_hbm, out_hbm,
          idxs, idxd, rows, acc, hn_sh,
          semz, semh, semi0, semi1, sg, ss):
        c = lax.axis_index("c")
        s = lax.axis_index("s")
        w = c * NTILE + s
        r0 = s * ROWS_PER_TILE
        hrows = N // NTILE
        zc = pltpu.async_copy(z_hbm.at[pl.ds(r0, ROWS_PER_TILE)],
                              acc.at[pl.ds(r0, ROWS_PER_TILE)], semz)
        hc = pltpu.async_copy(hn_hbm.at[pl.ds(s * hrows, hrows)],
                              hn_sh.at[pl.ds(s * hrows, hrows)], semh)
        ic0 = pltpu.async_copy(src_hbm.at[w], idxs, semi0)
        ic1 = pltpu.async_copy(dst_hbm.at[w], idxd, semi1)
        zc.wait()
        hc.wait()
        plsc.subcore_barrier()
        ic0.wait()
        ic1.wait()

        def g(kk, b):
            pltpu.async_copy(hn_sh.at[idxs.at[kk]], rows[b], sg[b])

        def gwait(kk, b):
            pltpu.make_async_copy(hn_sh.at[idxs.at[kk]], rows[b], sg[b]).wait()

        def sca(kk, b):
            pltpu.async_copy(rows[b], acc.at[idxd.at[kk]], ss[b], add=True)

        def swait(kk, b):
            pltpu.make_async_copy(rows[b], acc.at[idxd.at[kk]], ss[b]).wait()

        for b in range(NB):
            g(b, b)

        def step(j, carry):
            k0 = NB * j
            for b in range(NB):
                gwait(k0 + b, b)
                sca(k0 + b, b)
            for b in range(NB):
                swait(k0 + b, b)
                g(k0 + NB + b, b)
            return carry

        lax.fori_loop(0, CPW // NB - 1, step, 0)
        k0 = CPW - NB
        for b in range(NB):
            gwait(k0 + b, b)
            sca(k0 + b, b)
        for b in range(NB):
            swait(k0 + b, b)

        plsc.subcore_barrier()
        pltpu.sync_copy(acc.at[pl.ds(r0, ROWS_PER_TILE)],
                        out_hbm.at[c, pl.ds(r0, ROWS_PER_TILE)])

    return k(hn, src3, dst3, zb)


NROWS = E // CHUNK
NROWS_PAD = NW * CPW - NROWS


def _ln(h, g, b):
    mu = jnp.mean(h, axis=-1, keepdims=True)
    var = jnp.mean((h - mu) ** 2, axis=-1, keepdims=True)
    return (h - mu) / jnp.sqrt(var + 1e-5) * g + b


def _pad_idx_body(ei_ref, so_ref, do_ref):
    pad_s = jnp.zeros((NROWS_PAD, CHUNK), jnp.int32)
    pad_d = jnp.full((NROWS_PAD, CHUNK), N, jnp.int32)
    so_ref[...] = jnp.concatenate([ei_ref[0], pad_s], axis=0)
    do_ref[...] = jnp.concatenate([ei_ref[1], pad_d], axis=0)


def _bea_body(eap_ref, degp_ref, wme_ref, bm_ref, o0_ref, o1_ref, o2_ref):
    ea = eap_ref[0, :N] + eap_ref[1, :N]
    deg = degp_ref[...]
    for i, o_ref in enumerate((o0_ref, o1_ref, o2_ref)):
        o_ref[...] = (jnp.dot(ea, wme_ref[i],
                              preferred_element_type=jnp.float32)
                      + deg * bm_ref[i])


def _t0_body(hnb_ref, ws_ref, bs_ref, bea_ref, o_ref):
    o_ref[...] = (jnp.dot(hnb_ref[...], ws_ref[...],
                          preferred_element_type=jnp.float32)
                  + bs_ref[...] + bea_ref[...])


def _layer_update(h_ref, t_ref, gp_ref, wmh_ref):
    g = (gp_ref[0, :N].astype(jnp.float32)
         + gp_ref[1, :N].astype(jnp.float32))
    up = t_ref[...] + jnp.dot(g, wmh_ref[...],
                              preferred_element_type=jnp.float32)
    return h_ref[...] + jnp.maximum(up, 0.0)


def _first_body(x_ref, w_ref, b_ref, g_ref, be_ref, h_ref, hnb_ref):
    h = jnp.dot(x_ref[...], w_ref[...],
                preferred_element_type=jnp.float32) + b_ref[...]
    h_ref[...] = h
    hnb_ref[...] = _ln(h, g_ref[...], be_ref[...]).astype(jnp.bfloat16)


def _upd_body(h_ref, t_ref, gp_ref, wmh_ref, g2_ref, be2_ref,
              ws2_ref, bs2_ref, bea2_ref,
              ho_ref, hnbo_ref, t2_ref):
    h = _layer_update(h_ref, t_ref, gp_ref, wmh_ref)
    ho_ref[...] = h
    hn = _ln(h, g2_ref[...], be2_ref[...])
    hnbo_ref[...] = hn.astype(jnp.bfloat16)
    t2_ref[...] = (jnp.dot(hn, ws2_ref[...],
                           preferred_element_type=jnp.float32)
                   + bs2_ref[...] + bea2_ref[...])


def _final_body(h_ref, t_ref, gp_ref, wmh_ref, batch_ref, w1_ref, b1_ref,
                w2_ref, b2_ref, o_ref):
    h = _layer_update(h_ref, t_ref, gp_ref, wmh_ref)
    gids = lax.broadcasted_iota(jnp.int32, (1, NUM_GRAPHS), 1)
    onehot = (batch_ref[...] == gids).astype(jnp.float32)
    pooled = lax.dot_general(onehot, h, (((0,), (0,)), ((), ())),
                             preferred_element_type=jnp.float32)
    counts = lax.dot_general(onehot, jnp.ones((N, 1), jnp.float32),
                             (((0,), (0,)), ((), ())),
                             preferred_element_type=jnp.float32)
    pooled = pooled / jnp.maximum(counts, 1.0)
    e = jnp.maximum(
        jnp.dot(pooled, w1_ref[...], preferred_element_type=jnp.float32)
        + b1_ref[...], 0.0)
    e = jnp.dot(e, w2_ref[...], preferred_element_type=jnp.float32) + b2_ref[...]
    norm = jnp.sqrt(jnp.sum(e * e, axis=-1, keepdims=True))
    o_ref[...] = e / jnp.maximum(norm, 1e-12)


def _tc(body, out_shape, *args):
    return pl.pallas_call(body, out_shape=out_shape)(*args)


def kernel(x, edge_index, edge_attr, batch, Win, bin_, gamma, beta,
           Wm, bm, Ws, bs, W1, b1, W2, b2):
    f32 = jnp.float32
    ei3 = edge_index.reshape(2, NROWS, CHUNK)
    src2, dst2 = _tc(
        _pad_idx_body,
        [jax.ShapeDtypeStruct((NW * CPW, CHUNK), jnp.int32),
         jax.ShapeDtypeStruct((NW * CPW, CHUNK), jnp.int32)],
        ei3)
    src3 = src2.reshape(NW, CPW, CHUNK)
    dst3 = dst2.reshape(NW, CPW, CHUNK)
    zb = jnp.zeros((NPAD, H), jnp.bfloat16)
    z16 = jnp.zeros((NPAD, D_EDGE), f32)
    z1 = jnp.zeros((NPAD,), f32)
    ones_c = jnp.ones((CHUNK,), f32)

    eap_pad, degp_pad = _sc_edge_pre(edge_attr, dst3, z16, z1, ones_c)
    degp = (degp_pad[0, :N] + degp_pad[1, :N]).reshape(N, 1)

    h, hnb = _tc(
        _first_body,
        [jax.ShapeDtypeStruct((N, H), f32),
         jax.ShapeDtypeStruct((N, H), jnp.bfloat16)],
        x, Win, bin_.reshape(1, H),
        gamma[0].reshape(1, H), beta[0].reshape(1, H))

    nh = jax.ShapeDtypeStruct((N, H), f32)
    bea = _tc(_bea_body, [nh, nh, nh],
              eap_pad, degp, Wm[:, H:, :], bm.reshape(L, 1, H))

    t = _tc(_t0_body, nh, hnb, Ws[0], bs[0].reshape(1, H), bea[0])

    for i in range(L):
        wm_h = Wm[i][:H]
        gp = _sc_agg(hnb, src3, dst3, zb)
        if i < L - 1:
            h, hnb, t = _tc(
                _upd_body,
                [nh, jax.ShapeDtypeStruct((N, H), jnp.bfloat16), nh],
                h, t, gp, wm_h,
                gamma[i + 1].reshape(1, H), beta[i + 1].reshape(1, H),
                Ws[i + 1], bs[i + 1].reshape(1, H), bea[i + 1])
        else:
            out = _tc(
                _final_body, jax.ShapeDtypeStruct((NUM_GRAPHS, EMB), f32),
                h, t, gp, wm_h,
                batch.reshape(N, 1), W1, b1.reshape(1, EMB),
                W2, b2.reshape(1, EMB))
    return out
